# baseline (speedup 1.0000x reference)
import functools

import jax
import jax.numpy as jnp
from jax import lax
from jax.experimental import pallas as pl
from jax.experimental.pallas import tpu as pltpu
from jax.experimental.pallas import tpu_sc as plsc

N = 10000
E = 320000
D = 128
H = 128

NC = 2
NS = 16
NW = NC * NS
CHUNK = 128
NCHUNK = 80
GSZ = 16
NG = NCHUNK // GSZ
GE = GSZ * CHUNK
NP = 10240
ROWS_PER_SUB = NP // NS
EPW = NCHUNK * CHUNK
E_PAD = NW * EPW


def _sc_aggregate(x, zrows, src_idx, dst_idx):
    mesh = plsc.VectorSubcoreMesh(core_axis_name="c", subcore_axis_name="s")

    @functools.partial(
        pl.kernel,
        mesh=mesh,
        out_type=jax.ShapeDtypeStruct((NC, NP, D), jnp.float32),
        scratch_types=[
            pltpu.VMEM((2, GE), jnp.int32),
            pltpu.VMEM((2, GSZ, CHUNK), jnp.int32),
            pltpu.VMEM((CHUNK, D), jnp.float32),
            pltpu.VMEM((CHUNK, D), jnp.float32),
            pltpu.VMEM_SHARED((NP, D), jnp.float32),
            pltpu.SemaphoreType.DMA,
            pltpu.SemaphoreType.DMA,
            pltpu.SemaphoreType.DMA,
        ],
    )
    def sc_kernel(x_hbm, z_hbm, src_hbm, dst_hbm, out_hbm,
                  src_g, dst_g, rows0, rows1, agg_sh, gsem, ssem, isem):
        c = lax.axis_index("c")
        s = lax.axis_index("s")
        wid = c * NS + s
        rbufs = (rows0, rows1)
        row0 = s * ROWS_PER_SUB

        pltpu.sync_copy(z_hbm, rows0)
        for k in range(ROWS_PER_SUB // CHUNK):
            pltpu.async_copy(rows0, agg_sh.at[pl.ds(row0 + k * CHUNK, CHUNK)],
                             ssem)
        for k in range(ROWS_PER_SUB // CHUNK):
            pltpu.make_async_copy(rows0,
                                  agg_sh.at[pl.ds(row0, CHUNK)], ssem).wait()
        plsc.subcore_barrier()

        def load_idx(g, p):
            pltpu.async_copy(src_hbm.at[pl.ds(wid * EPW + g * GE, GE)],
                             src_g.at[p], isem)
            pltpu.async_copy(dst_hbm.at[wid, pl.ds(g * GSZ, GSZ)],
                             dst_g.at[p], isem)

        def wait_idx(p):
            pltpu.make_async_copy(src_hbm.at[pl.ds(0, GE)],
                                  src_g.at[p], isem).wait()
            pltpu.make_async_copy(dst_hbm.at[pl.ds(0, 1), pl.ds(0, GSZ)],
                                  dst_g.at[p], isem).wait()

        def start_gather(idx_row, rbuf):
            pltpu.async_copy(x_hbm.at[idx_row], rbuf, gsem)

        def wait_gather(rbuf):
            pltpu.make_async_copy(x_hbm.at[pl.ds(0, CHUNK)], rbuf, gsem).wait()

        def start_scat(rbuf, idx_row):
            pltpu.async_copy(rbuf, agg_sh.at[idx_row], ssem, add=True)

        def wait_scat():
            pltpu.make_async_copy(rows0, agg_sh.at[pl.ds(0, CHUNK)],
                                  ssem).wait()

        def src_row(p, j):
            return src_g.at[p, pl.ds(j * CHUNK, CHUNK)]

        def chunk_step(next_src, dst_row, b):
            wait_gather(rbufs[b])
            start_scat(rbufs[b], dst_row)
            wait_scat()
            start_gather(next_src, rbufs[1 - b])

        load_idx(0, 0)
        wait_idx(0)
        start_gather(src_row(0, 0), rows0)
        wait_gather(rows0)
        start_scat(rows0, dst_g.at[0, 0])
        start_gather(src_row(0, 1), rows1)

        for g in range(NG):
            p = g % 2
            if g + 1 < NG:
                load_idx(g + 1, 1 - p)
            if g > 0:
                chunk_step(src_row(p, 1), dst_g.at[p, 0], 0)

            def pair_body(kk, carry, p=p):
                j = 2 * kk + 1
                chunk_step(src_row(p, j + 1), dst_g.at[p, j], 1)
                chunk_step(src_row(p, j + 2), dst_g.at[p, j + 1], 0)
                return carry

            lax.fori_loop(0, GSZ // 2 - 1, pair_body, 0)
            if g + 1 < NG:
                wait_idx(1 - p)
                chunk_step(src_row(1 - p, 0), dst_g.at[p, GSZ - 1], 1)
            else:
                wait_gather(rows1)
                start_scat(rows1, dst_g.at[p, GSZ - 1])
                wait_scat()
                wait_scat()

        plsc.subcore_barrier()

        def wait_out():
            pltpu.make_async_copy(rows0, out_hbm.at[c, pl.ds(row0, CHUNK)],
                                  gsem).wait()

        for k in range(ROWS_PER_SUB // CHUNK):
            b = k % 2
            r0 = row0 + k * CHUNK
            if k >= 2:
                wait_out()
            pltpu.sync_copy(agg_sh.at[pl.ds(r0, CHUNK)], rbufs[b])
            pltpu.async_copy(rbufs[b], out_hbm.at[c, pl.ds(r0, CHUNK)], gsem)
        wait_out()
        wait_out()

    return sc_kernel(x, zrows, src_idx, dst_idx)


def _mlp(x, parts, W1, b1, W2, b2):
    BR = 2000

    def body(x_ref, p_ref, w1_ref, b1_ref, w2_ref, b2_ref, o_ref):
        xx = x_ref[...]
        h = xx + p_ref[0] + p_ref[1]
        z = jnp.dot(h, w1_ref[...], preferred_element_type=jnp.float32)
        z = jnp.maximum(z + b1_ref[...], 0.0)
        o = jnp.dot(z, w2_ref[...], preferred_element_type=jnp.float32)
        o_ref[...] = o + b2_ref[...] + xx

    return pl.pallas_call(
        body,
        grid=(N // BR,),
        in_specs=[
            pl.BlockSpec((BR, D), lambda i: (i, 0)),
            pl.BlockSpec((NC, BR, D), lambda i: (0, i, 0)),
            pl.BlockSpec((D, H), lambda i: (0, 0)),
            pl.BlockSpec((1, H), lambda i: (0, 0)),
            pl.BlockSpec((H, D), lambda i: (0, 0)),
            pl.BlockSpec((1, D), lambda i: (0, 0)),
        ],
        out_specs=pl.BlockSpec((BR, D), lambda i: (i, 0)),
        out_shape=jax.ShapeDtypeStruct((N, D), jnp.float32),
    )(x, parts, W1, b1.reshape(1, H), W2, b2.reshape(1, D))


def kernel(x, edge_index, W1, b1, W2, b2):
    pad = E_PAD - E
    pad_dst = N + jnp.arange(pad, dtype=jnp.int32) % (NP - N)
    src = jnp.concatenate([edge_index[0], jnp.zeros((pad,), jnp.int32)])
    dst = jnp.concatenate([edge_index[1], pad_dst]).reshape(NW, NCHUNK, CHUNK)
    zrows = jnp.zeros((CHUNK, D), jnp.float32)
    parts = _sc_aggregate(x, zrows, src, dst)
    return _mlp(x, parts, W1, b1, W2, b2)


# speedup vs baseline: 1.0030x; 1.0030x over previous
import functools

import jax
import jax.numpy as jnp
from jax import lax
from jax.experimental import pallas as pl
from jax.experimental.pallas import tpu as pltpu
from jax.experimental.pallas import tpu_sc as plsc

N = 10000
E = 320000
D = 128
H = 128

NC = 2
NS = 16
NW = NC * NS
CHUNK = 128
NCHUNK = 80
GSZ = 16
NG = NCHUNK // GSZ
GE = GSZ * CHUNK
NP = 10240
ROWS_PER_SUB = NP // NS
EPW = NCHUNK * CHUNK
E_PAD = NW * EPW


def _sc_aggregate(x, zrows, src_idx, dst_idx):
    mesh = plsc.VectorSubcoreMesh(core_axis_name="c", subcore_axis_name="s")

    @functools.partial(
        pl.kernel,
        mesh=mesh,
        out_type=jax.ShapeDtypeStruct((NC, NP, D), jnp.float32),
        scratch_types=[
            pltpu.VMEM((2, GSZ, CHUNK), jnp.int32),
            pltpu.VMEM((2, GSZ, CHUNK), jnp.int32),
            pltpu.VMEM((CHUNK, D), jnp.float32),
            pltpu.VMEM((CHUNK, D), jnp.float32),
            pltpu.VMEM_SHARED((NP, D), jnp.float32),
            pltpu.SemaphoreType.DMA,
            pltpu.SemaphoreType.DMA,
            pltpu.SemaphoreType.DMA,
        ],
    )
    def sc_kernel(x_hbm, z_hbm, src_hbm, dst_hbm, out_hbm,
                  src_g, dst_g, rows0, rows1, agg_sh, gsem, ssem, isem):
        c = lax.axis_index("c")
        s = lax.axis_index("s")
        wid = c * NS + s
        rbufs = (rows0, rows1)
        row0 = s * ROWS_PER_SUB

        pltpu.sync_copy(z_hbm, rows0)
        for k in range(ROWS_PER_SUB // CHUNK):
            pltpu.async_copy(rows0, agg_sh.at[pl.ds(row0 + k * CHUNK, CHUNK)],
                             ssem)
        for k in range(ROWS_PER_SUB // CHUNK):
            pltpu.make_async_copy(rows0,
                                  agg_sh.at[pl.ds(row0, CHUNK)], ssem).wait()
        plsc.subcore_barrier()

        def load_idx(g, p):
            pltpu.async_copy(src_hbm.at[wid, pl.ds(g * GSZ, GSZ)],
                             src_g.at[p], isem)
            pltpu.async_copy(dst_hbm.at[wid, pl.ds(g * GSZ, GSZ)],
                             dst_g.at[p], isem)

        def wait_idx(p):
            pltpu.make_async_copy(src_hbm.at[pl.ds(0, 1), pl.ds(0, GSZ)],
                                  src_g.at[p], isem).wait()
            pltpu.make_async_copy(dst_hbm.at[pl.ds(0, 1), pl.ds(0, GSZ)],
                                  dst_g.at[p], isem).wait()

        def start_gather(idx_row, rbuf):
            pltpu.async_copy(x_hbm.at[idx_row], rbuf, gsem)

        def wait_gather(rbuf):
            pltpu.make_async_copy(x_hbm.at[pl.ds(0, CHUNK)], rbuf, gsem).wait()

        def start_scat(rbuf, idx_row):
            pltpu.async_copy(rbuf, agg_sh.at[idx_row], ssem, add=True)

        def wait_scat():
            pltpu.make_async_copy(rows0, agg_sh.at[pl.ds(0, CHUNK)],
                                  ssem).wait()

        def src_row(p, j):
            return src_g.at[p, j]

        def chunk_step(next_src, dst_row, b):
            wait_gather(rbufs[b])
            start_scat(rbufs[b], dst_row)
            wait_scat()
            start_gather(next_src, rbufs[1 - b])

        load_idx(0, 0)
        wait_idx(0)
        start_gather(src_row(0, 0), rows0)
        wait_gather(rows0)
        start_scat(rows0, dst_g.at[0, 0])
        start_gather(src_row(0, 1), rows1)

        for g in range(NG):
            p = g % 2
            if g + 1 < NG:
                load_idx(g + 1, 1 - p)
            if g > 0:
                chunk_step(src_row(p, 1), dst_g.at[p, 0], 0)

            def pair_body(kk, carry, p=p):
                j = 2 * kk + 1
                chunk_step(src_row(p, j + 1), dst_g.at[p, j], 1)
                chunk_step(src_row(p, j + 2), dst_g.at[p, j + 1], 0)
                return carry

            lax.fori_loop(0, GSZ // 2 - 1, pair_body, 0)
            if g + 1 < NG:
                wait_idx(1 - p)
                chunk_step(src_row(1 - p, 0), dst_g.at[p, GSZ - 1], 1)
            else:
                wait_gather(rows1)
                start_scat(rows1, dst_g.at[p, GSZ - 1])
                wait_scat()
                wait_scat()

        plsc.subcore_barrier()

        def wait_out():
            pltpu.make_async_copy(rows0, out_hbm.at[c, pl.ds(row0, CHUNK)],
                                  gsem).wait()

        for k in range(ROWS_PER_SUB // CHUNK):
            b = k % 2
            r0 = row0 + k * CHUNK
            if k >= 2:
                wait_out()
            pltpu.sync_copy(agg_sh.at[pl.ds(r0, CHUNK)], rbufs[b])
            pltpu.async_copy(rbufs[b], out_hbm.at[c, pl.ds(r0, CHUNK)], gsem)
        wait_out()
        wait_out()

    return sc_kernel(x, zrows, src_idx, dst_idx)


def _mlp(x, parts, W1, b1, W2, b2):
    BR = 2000

    def body(x_ref, p_ref, w1_ref, b1_ref, w2_ref, b2_ref, o_ref):
        xx = x_ref[...]
        h = xx + p_ref[0] + p_ref[1]
        z = jnp.dot(h, w1_ref[...], preferred_element_type=jnp.float32)
        z = jnp.maximum(z + b1_ref[...], 0.0)
        o = jnp.dot(z, w2_ref[...], preferred_element_type=jnp.float32)
        o_ref[...] = o + b2_ref[...] + xx

    return pl.pallas_call(
        body,
        grid=(N // BR,),
        in_specs=[
            pl.BlockSpec((BR, D), lambda i: (i, 0)),
            pl.BlockSpec((NC, BR, D), lambda i: (0, i, 0)),
            pl.BlockSpec((D, H), lambda i: (0, 0)),
            pl.BlockSpec((1, H), lambda i: (0, 0)),
            pl.BlockSpec((H, D), lambda i: (0, 0)),
            pl.BlockSpec((1, D), lambda i: (0, 0)),
        ],
        out_specs=pl.BlockSpec((BR, D), lambda i: (i, 0)),
        out_shape=jax.ShapeDtypeStruct((N, D), jnp.float32),
    )(x, parts, W1, b1.reshape(1, H), W2, b2.reshape(1, D))


def kernel(x, edge_index, W1, b1, W2, b2):
    pad = E_PAD - E
    pad_dst = N + jnp.arange(pad, dtype=jnp.int32) % (NP - N)
    src = jnp.concatenate(
        [edge_index[0], jnp.zeros((pad,), jnp.int32)]).reshape(NW, NCHUNK, CHUNK)
    dst = jnp.concatenate([edge_index[1], pad_dst]).reshape(NW, NCHUNK, CHUNK)
    zrows = jnp.zeros((CHUNK, D), jnp.float32)
    parts = _sc_aggregate(x, zrows, src, dst)
    return _mlp(x, parts, W1, b1, W2, b2)


# speedup vs baseline: 3.0407x; 3.0317x over previous
import functools

import jax
import jax.numpy as jnp
from jax import lax
from jax.experimental import pallas as pl
from jax.experimental.pallas import tpu as pltpu
from jax.experimental.pallas import tpu_sc as plsc

N = 10000
E = 320000
D = 128
H = 128

NC = 2
NS = 16
NW = NC * NS
CHUNK = 128
NCHUNK = 80
GSZ = 16
NG = NCHUNK // GSZ
GE = GSZ * CHUNK
NP = 10240
ROWS_PER_SUB = NP // NS
EPW = NCHUNK * CHUNK
E_PAD = NW * EPW


def _sc_aggregate(x, zrows, src_idx, dst_idx):
    mesh = plsc.VectorSubcoreMesh(core_axis_name="c", subcore_axis_name="s")

    @functools.partial(
        pl.kernel,
        mesh=mesh,
        out_type=jax.ShapeDtypeStruct((NC, NP, D), jnp.float32),
        scratch_types=[
            pltpu.VMEM((2, GSZ, CHUNK), jnp.int32),
            pltpu.VMEM((2, GSZ, CHUNK), jnp.int32),
            pltpu.VMEM((CHUNK, D), jnp.float32),
            pltpu.VMEM((CHUNK, D), jnp.float32),
            pltpu.VMEM_SHARED((NP, D), jnp.float32),
            pltpu.SemaphoreType.DMA,
            pltpu.SemaphoreType.DMA,
            pltpu.SemaphoreType.DMA,
        ],
    )
    def sc_kernel(x_hbm, z_hbm, src_hbm, dst_hbm, out_hbm,
                  src_g, dst_g, rows0, rows1, agg_sh, gsem, ssem, isem):
        c = lax.axis_index("c")
        s = lax.axis_index("s")
        wid = c * NS + s
        rbufs = (rows0, rows1)
        row0 = s * ROWS_PER_SUB

        pltpu.sync_copy(z_hbm, rows0)
        for k in range(ROWS_PER_SUB // CHUNK):
            pltpu.async_copy(rows0, agg_sh.at[pl.ds(row0 + k * CHUNK, CHUNK)],
                             ssem)
        for k in range(ROWS_PER_SUB // CHUNK):
            pltpu.make_async_copy(rows0,
                                  agg_sh.at[pl.ds(row0, CHUNK)], ssem).wait()
        plsc.subcore_barrier()

        def load_idx(g, p):
            pltpu.async_copy(src_hbm.at[wid, pl.ds(g * GSZ, GSZ)],
                             src_g.at[p], isem)
            pltpu.async_copy(dst_hbm.at[wid, pl.ds(g * GSZ, GSZ)],
                             dst_g.at[p], isem)

        def wait_idx(p):
            pltpu.make_async_copy(src_hbm.at[pl.ds(0, 1), pl.ds(0, GSZ)],
                                  src_g.at[p], isem).wait()
            pltpu.make_async_copy(dst_hbm.at[pl.ds(0, 1), pl.ds(0, GSZ)],
                                  dst_g.at[p], isem).wait()

        def start_gather(idx_row, rbuf):
            pltpu.async_copy(x_hbm.at[idx_row], rbuf, gsem)

        def wait_gather(rbuf):
            pltpu.make_async_copy(x_hbm.at[pl.ds(0, CHUNK)], rbuf, gsem).wait()

        def start_scat(rbuf, idx_row):
            pltpu.async_copy(rbuf, agg_sh.at[idx_row], ssem, add=True)

        def wait_scat():
            pltpu.make_async_copy(rows0, agg_sh.at[pl.ds(0, CHUNK)],
                                  ssem).wait()

        def src_row(p, j):
            return src_g.at[p, j]

        def chunk_step(next_src, dst_row, b):
            wait_gather(rbufs[b])
            start_scat(rbufs[b], dst_row)
            wait_scat()
            start_gather(next_src, rbufs[1 - b])

        load_idx(0, 0)
        wait_idx(0)
        start_gather(src_row(0, 0), rows0)
        wait_gather(rows0)
        start_scat(rows0, dst_g.at[0, 0])
        start_gather(src_row(0, 1), rows1)

        for g in range(NG):
            p = g % 2
            if g + 1 < NG:
                load_idx(g + 1, 1 - p)
            if g > 0:
                chunk_step(src_row(p, 1), dst_g.at[p, 0], 0)

            def pair_body(kk, carry, p=p):
                j = 2 * kk + 1
                chunk_step(src_row(p, j + 1), dst_g.at[p, j], 1)
                chunk_step(src_row(p, j + 2), dst_g.at[p, j + 1], 0)
                return carry

            lax.fori_loop(0, GSZ // 2 - 1, pair_body, 0)
            if g + 1 < NG:
                wait_idx(1 - p)
                chunk_step(src_row(1 - p, 0), dst_g.at[p, GSZ - 1], 1)
            else:
                wait_gather(rows1)
                start_scat(rows1, dst_g.at[p, GSZ - 1])
                wait_scat()
                wait_scat()

        plsc.subcore_barrier()

        def wait_out():
            pltpu.make_async_copy(rows0, out_hbm.at[c, pl.ds(row0, CHUNK)],
                                  gsem).wait()

        for k in range(ROWS_PER_SUB // CHUNK):
            b = k % 2
            r0 = row0 + k * CHUNK
            if k >= 2:
                wait_out()
            pltpu.sync_copy(agg_sh.at[pl.ds(r0, CHUNK)], rbufs[b])
            pltpu.async_copy(rbufs[b], out_hbm.at[c, pl.ds(r0, CHUNK)], gsem)
        wait_out()
        wait_out()

    return sc_kernel(x, zrows, src_idx, dst_idx)


def _mlp(x, parts, W1, b1, W2, b2):
    BR = 2000

    def body(x_ref, p_ref, w1_ref, b1_ref, w2_ref, b2_ref, o_ref):
        xx = x_ref[...]
        h = xx + p_ref[0] + p_ref[1]
        z = jnp.dot(h, w1_ref[...], preferred_element_type=jnp.float32)
        z = jnp.maximum(z + b1_ref[...], 0.0)
        o = jnp.dot(z, w2_ref[...], preferred_element_type=jnp.float32)
        o_ref[...] = o + b2_ref[...] + xx

    return pl.pallas_call(
        body,
        grid=(N // BR,),
        in_specs=[
            pl.BlockSpec((BR, D), lambda i: (i, 0)),
            pl.BlockSpec((NC, BR, D), lambda i: (0, i, 0)),
            pl.BlockSpec((D, H), lambda i: (0, 0)),
            pl.BlockSpec((1, H), lambda i: (0, 0)),
            pl.BlockSpec((H, D), lambda i: (0, 0)),
            pl.BlockSpec((1, D), lambda i: (0, 0)),
        ],
        out_specs=pl.BlockSpec((BR, D), lambda i: (i, 0)),
        out_shape=jax.ShapeDtypeStruct((N, D), jnp.float32),
    )(x, parts, W1, b1.reshape(1, H), W2, b2.reshape(1, D))


def kernel(x, edge_index, W1, b1, W2, b2):
    pad = E_PAD - E
    pad_dst = N + jnp.arange(pad, dtype=jnp.int32) % (NP - N)
    pad_src = jnp.arange(pad, dtype=jnp.int32) % N
    src = jnp.concatenate(
        [edge_index[0], pad_src]).reshape(NW, NCHUNK, CHUNK)
    dst = jnp.concatenate([edge_index[1], pad_dst]).reshape(NW, NCHUNK, CHUNK)
    zrows = jnp.zeros((CHUNK, D), jnp.float32)
    parts = _sc_aggregate(x, zrows, src, dst)
    return _mlp(x, parts, W1, b1, W2, b2)
